# baseline (speedup 1.0000x reference)
import functools

import jax
import jax.numpy as jnp
from jax import lax
from jax.experimental import pallas as pl
from jax.experimental.pallas import tpu as pltpu
from jax.experimental.pallas import tpu_sc as plsc

NUM_EMB = 1_000_000
DIM = 64
SCALE = 8.0

NC = 2
NS = 16
NW = NC * NS
B = 4096 * 200
W = B // NW
SUBLEN = 128
SUB = 5
C = SUB * SUBLEN
NCHUNK = W // C
IDXROWS = W // SUBLEN


def _body(idx_hbm, table_hbm, out_hbm, idx_v, rows_v, gsem, ssem):
  cid = lax.axis_index("c")
  sid = lax.axis_index("s")
  wid = sid * NC + cid

  pltpu.sync_copy(idx_hbm.at[pl.ds(wid * IDXROWS, IDXROWS)], idx_v)

  row_base = wid * W

  def fire_gather(g, b):
    for k in range(SUB):
      pltpu.async_copy(
          table_hbm.at[idx_v.at[g * SUB + k]],
          rows_v.at[b, pl.ds(k * SUBLEN, SUBLEN)],
          gsem.at[b],
      )

  def wait_gather(b):
    pltpu.make_async_copy(
        out_hbm.at[pl.ds(0, C)], rows_v.at[b], gsem.at[b]
    ).wait()

  def fire_store(g, b):
    pltpu.async_copy(
        rows_v.at[b], out_hbm.at[pl.ds(row_base + g * C, C)], ssem.at[b]
    )

  def wait_store(b):
    pltpu.make_async_copy(
        rows_v.at[b], out_hbm.at[pl.ds(0, C)], ssem.at[b]
    ).wait()

  def scale_buf(b):
    @plsc.parallel_loop(0, C, unroll=8)
    def _(r):
      for c in range(DIM // 16):
        rows_v[b, r, pl.ds(c * 16, 16)] = (
            rows_v[b, r, pl.ds(c * 16, 16)] * SCALE
        )

  fire_gather(0, 0)

  @pl.loop(0, NCHUNK, step=2)
  def _(g):
    for b in range(2):
      gg = g + b
      @pl.when(gg + 1 < NCHUNK)
      def _():
        @pl.when(gg >= 1)
        def _():
          wait_store(1 - b)
        fire_gather(gg + 1, 1 - b)

      wait_gather(b)
      scale_buf(b)
      fire_store(gg, b)

  wait_store(0)
  wait_store(1)


@jax.jit
def _embed(idx2d, table):
  mesh = plsc.VectorSubcoreMesh(core_axis_name="c", subcore_axis_name="s")
  run = pl.kernel(
      _body,
      out_type=jax.ShapeDtypeStruct((B, DIM), jnp.float32),
      mesh=mesh,
      scratch_types=[
          pltpu.VMEM((IDXROWS, SUBLEN), jnp.int32),
          pltpu.VMEM((2, C, DIM), jnp.float32),
          pltpu.SemaphoreType.DMA((2,)),
          pltpu.SemaphoreType.DMA((2,)),
      ],
      compiler_params=pltpu.CompilerParams(use_tc_tiling_on_sc=False),
  )
  return run(idx2d, table)


def kernel(token_ids, tok_embedding):
  idx2d = token_ids.reshape(B // SUBLEN, SUBLEN).astype(jnp.int32)
  out = _embed(idx2d, tok_embedding)
  return out.reshape(token_ids.shape + (DIM,))
